# baseline (speedup 1.0000x reference)
import functools

import jax
import jax.numpy as jnp
from jax import lax
from jax.experimental import pallas as pl
from jax.experimental.pallas import tpu as pltpu
from jax.experimental.pallas import tpu_sc as plsc

EMB = 64
BATCH = 4096
NROWS = 100000
L = 16
NC = 2
NS = 16
NW = NC * NS
PIECE = 4992
NPIECE = 10
HALF0 = NPIECE * PIECE
TAIL = NROWS - 2 * HALF0
NCH = BATCH // L
LISTN = BATCH + 16 * (NPIECE + 1)


def _sc_gather(u_t, f_t, ub1, fb1, uid, fid):
    mesh = plsc.VectorSubcoreMesh(core_axis_name="c", subcore_axis_name="s")

    @functools.partial(
        pl.kernel,
        mesh=mesh,
        compiler_params=pltpu.CompilerParams(needs_layout_passes=False),
        out_type=(
            jax.ShapeDtypeStruct((16, 8, BATCH), jnp.float32),
            jax.ShapeDtypeStruct((16, 8, BATCH), jnp.float32),
            jax.ShapeDtypeStruct((2, 8, BATCH), jnp.float32),
            jax.ShapeDtypeStruct((2, 8, BATCH), jnp.float32),
        ),
        scratch_types=[
            pltpu.VMEM((BATCH,), jnp.int32),
            pltpu.VMEM((8, PIECE), jnp.float32),
            pltpu.VMEM((8, PIECE), jnp.float32),
            pltpu.VMEM((8, BATCH), jnp.float32),
            pltpu.VMEM((LISTN,), jnp.int32),
            pltpu.VMEM((LISTN,), jnp.int32),
            pltpu.VMEM((PIECE,), jnp.float32),
            pltpu.VMEM((TAIL,), jnp.float32),
            pltpu.SemaphoreType.DMA,
            pltpu.SemaphoreType.DMA,
        ],
    )
    def k(ut_h, ft_h, ub_h, fb_h, uid_h, fid_h,
          uout, fout, ubg_out, fbg_out,
          ids_v, pa_v, pb_v, stage_v, idl_v, posl_v, b1d_v, tail_v,
          sem_a, sem_b):
        wid = lax.axis_index("s") * NC + lax.axis_index("c")
        tbl = wid // 16
        rest = wid % 16
        tr = rest // 2
        h = rest % 2
        base = h * HALF0
        lane = jnp.arange(L, dtype=jnp.int32)
        zf = jnp.zeros((L,), jnp.float32)

        def zero_stage(c, _):
            for q in range(4):
                stage_v[(c * 4 + q) // 256, pl.ds(((c * 4 + q) % 256) * L, L)] = zf
            return 0

        def run(tab_h, bias_h, id_h, out_ref, bout_ref):
            pltpu.sync_copy(id_h, ids_v)
            lax.fori_loop(0, (8 * BATCH) // L // 4, zero_stage, 0)

            bufs = (pa_v, pb_v)
            sems = (sem_a, sem_b)

            def start_piece(p):
                lo = base + p * PIECE
                return pltpu.async_copy(
                    tab_h.at[pl.ds(tr * 8, 8), pl.ds(lo, PIECE)],
                    bufs[p % 2], sems[p % 2])

            def scan_piece(lo_off, size, seg):
                def body(c, cnt):
                    idx = ids_v[pl.ds(c * L, L)]
                    lidx = idx - lo_off
                    m = (idx >= lo_off) & (idx < (lo_off + size))
                    plsc.store_compressed(idl_v.at[pl.ds(seg + cnt, L)],
                                          lidx, mask=m)
                    plsc.store_compressed(posl_v.at[pl.ds(seg + cnt, L)],
                                          lane + c * L, mask=m)
                    mv = plsc.all_reduce_population_count(m)
                    return cnt + jnp.max(mv)

                return lax.fori_loop(0, NCH, body, jnp.int32(0))

            def gather_piece(buf, seg, cnt):
                def body(j, _):
                    lpos = j * L + lane
                    mv = lpos < cnt
                    lidxc = idl_v[pl.ds(seg + j * L, L)]
                    posc = posl_v[pl.ds(seg + j * L, L)]
                    for dd in range(8):
                        dvec = jnp.full((L,), dd, jnp.int32)
                        v = plsc.load_gather(buf, [dvec, lidxc], mask=mv)
                        plsc.store_scatter(stage_v, [dvec, posc], v, mask=mv)
                    return 0

                return lax.fori_loop(0, (cnt + L - 1) // L, body, 0)

            cp = start_piece(0)
            segs = []
            cnts = []
            seg = jnp.int32(0)
            for p in range(NPIECE):
                cnt = scan_piece(base + p * PIECE, PIECE, seg)
                cp.wait()
                if p + 1 < NPIECE:
                    cp = start_piece(p + 1)
                gather_piece(bufs[p % 2], seg, cnt)
                segs.append(seg)
                cnts.append(cnt)
                seg = seg + cnt

            @pl.when(h == 1)
            def _():
                lo = 2 * HALF0
                cnt_t = scan_piece(lo, TAIL, seg)
                for dd in range(8):
                    pltpu.sync_copy(
                        tab_h.at[tr * 8 + dd, pl.ds(lo, TAIL)], tail_v)
                    dvec = jnp.full((L,), dd, jnp.int32)

                    def tbody(j, _):
                        lpos = j * L + lane
                        mv = lpos < cnt_t
                        lidxc = idl_v[pl.ds(seg + j * L, L)]
                        posc = posl_v[pl.ds(seg + j * L, L)]
                        v = plsc.load_gather(tail_v, [lidxc], mask=mv)
                        plsc.store_scatter(stage_v, [dvec, posc], v, mask=mv)
                        return 0

                    lax.fori_loop(0, (cnt_t + L - 1) // L, tbody, 0)

            pltpu.sync_copy(stage_v, out_ref.at[tr * 2 + h])

            @pl.when(tr == 0)
            def _():
                def zrow(c, _):
                    for q in range(4):
                        stage_v[0, pl.ds((c * 4 + q) * L, L)] = zf
                    return 0

                lax.fori_loop(0, NCH // 4, zrow, 0)
                d0 = jnp.zeros((L,), jnp.int32)
                for p in range(NPIECE):
                    lo = base + p * PIECE
                    pltpu.sync_copy(bias_h.at[pl.ds(lo, PIECE)], b1d_v)

                    def bbody(j, _):
                        lpos = j * L + lane
                        mv = lpos < cnts[p]
                        lidxc = idl_v[pl.ds(segs[p] + j * L, L)]
                        posc = posl_v[pl.ds(segs[p] + j * L, L)]
                        v = plsc.load_gather(b1d_v, [lidxc], mask=mv)
                        plsc.store_scatter(stage_v, [d0, posc], v, mask=mv)
                        return 0

                    lax.fori_loop(0, (cnts[p] + L - 1) // L, bbody, 0)

                @pl.when(h == 1)
                def _():
                    lo = 2 * HALF0
                    pltpu.sync_copy(bias_h.at[pl.ds(lo, TAIL)], tail_v)
                    cnt_t = scan_piece(lo, TAIL, seg)

                    def bbody2(j, _):
                        lpos = j * L + lane
                        mv = lpos < cnt_t
                        lidxc = idl_v[pl.ds(seg + j * L, L)]
                        posc = posl_v[pl.ds(seg + j * L, L)]
                        v = plsc.load_gather(tail_v, [lidxc], mask=mv)
                        plsc.store_scatter(stage_v, [d0, posc], v, mask=mv)
                        return 0

                    lax.fori_loop(0, (cnt_t + L - 1) // L, bbody2, 0)

                pltpu.sync_copy(stage_v, bout_ref.at[h])

        @pl.when(tbl == 0)
        def _():
            run(ut_h, ub_h, uid_h, uout, ubg_out)

        @pl.when(tbl == 1)
        def _():
            run(ft_h, fb_h, fid_h, fout, fbg_out)

    return k(u_t, f_t, ub1, fb1, uid, fid)


def _tc_mlp(ublk, fblk, ubg, fbg, w1r, b1r, w2, b2r, w3r, b3r):
    def body(u_ref, f_ref, ub_ref, fb_ref, w1_ref, b1_ref, w2_ref, b2_ref,
             w3_ref, b3_ref, out_ref):
        u4 = u_ref[...].reshape(8, 2, 8, BATCH)
        f4 = f_ref[...].reshape(8, 2, 8, BATCH)
        uv = u4[:, 0] + u4[:, 1]
        fv = f4[:, 0] + f4[:, 1]
        s = jnp.sum(uv * fv)
        ub = (ub_ref[0, 0, :] + ub_ref[1, 0, :]).reshape(BATCH, 1)
        fb = (fb_ref[0, 0, :] + fb_ref[1, 0, :]).reshape(BATCH, 1)
        x = s + ub + fb
        h1 = jnp.maximum(x * w1_ref[...] + b1_ref[...], 0.0)
        h2 = jnp.maximum(
            jnp.dot(h1, w2_ref[...], preferred_element_type=jnp.float32)
            + b2_ref[...], 0.0)
        zz = jnp.sum(h2 * w3_ref[...], axis=1, keepdims=True) + b3_ref[...]
        out_ref[...] = 1.0 / (1.0 + jnp.exp(-zz))

    return pl.pallas_call(
        body,
        out_shape=jax.ShapeDtypeStruct((BATCH, 1), jnp.float32),
    )(ublk, fblk, ubg, fbg, w1r, b1r, w2, b2r, w3r, b3r)


def kernel(inputs, user_emb, user_bias, food_emb, food_bias, W1, b1, W2, b2, W3, b3):
    idx = inputs.astype(jnp.int32)
    uid = idx[:, 0]
    fid = idx[:, 1]
    ublk, fblk, ubg, fbg = _sc_gather(
        user_emb.T, food_emb.T,
        user_bias.reshape(-1), food_bias.reshape(-1),
        uid, fid)
    return _tc_mlp(
        ublk, fblk, ubg, fbg,
        W1.reshape(1, 128), b1.reshape(1, 128),
        W2, b2.reshape(1, 64),
        W3.reshape(1, 64), b3.reshape(1, 1))
